# Initial kernel scaffold; baseline (speedup 1.0000x reference)
import jax
import jax.numpy as jnp
from jax.experimental import pallas as pl


def kernel(x, token_table, pos_table):
    raise NotImplementedError("write your pallas kernel here")


# speedup vs baseline: 3.0628x; 3.0628x over previous
import jax
import jax.numpy as jnp
from jax import lax
from jax.experimental import pallas as pl
from jax.experimental.pallas import tpu as pltpu
from jax.experimental.pallas import tpu_sc as plsc

_BATCH = 4096
_SEQ = 200
_EMBED = 64
_LANES = 16
_NC = 2
_NS = 16
_NW = _NC * _NS
_ROWS_PER_W = _BATCH // _NW
_HALF = _SEQ // 2


def _sc_body(x_hbm, tok_hbm, pos_hbm, out_hbm, idx_v, rows_v, pos_v, sem):
    wid = lax.axis_index("s") * _NC + lax.axis_index("c")
    base = wid * _ROWS_PER_W
    pltpu.sync_copy(pos_hbm, pos_v)

    def row_body(r, carry):
        row = base + r
        pltpu.sync_copy(x_hbm.at[row], idx_v)
        cp0 = pltpu.async_copy(
            tok_hbm.at[idx_v.at[0]], rows_v.at[pl.ds(0, _HALF)], sem)
        cp1 = pltpu.async_copy(
            tok_hbm.at[idx_v.at[1]], rows_v.at[pl.ds(_HALF, _HALF)], sem)
        cp0.wait()
        cp1.wait()

        def add_body(i, c2):
            for d in range(_EMBED // _LANES):
                sl = pl.ds(d * _LANES, _LANES)
                rows_v[i, sl] = rows_v[i, sl] + pos_v[i, sl]
            return c2

        lax.fori_loop(0, _SEQ, add_body, 0)
        pltpu.sync_copy(rows_v, out_hbm.at[row])
        return carry

    lax.fori_loop(0, _ROWS_PER_W, row_body, 0)


def kernel(x, token_table, pos_table):
    x32 = x.astype(jnp.int32).reshape(_BATCH, 2, _HALF)
    f = pl.kernel(
        _sc_body,
        mesh=plsc.VectorSubcoreMesh(core_axis_name="c", subcore_axis_name="s"),
        compiler_params=pltpu.CompilerParams(use_tc_tiling_on_sc=False),
        out_type=jax.ShapeDtypeStruct((_BATCH, _SEQ, _EMBED), jnp.float32),
        scratch_types=[
            pltpu.VMEM((2, _HALF), jnp.int32),
            pltpu.VMEM((_SEQ, _EMBED), jnp.float32),
            pltpu.VMEM((_SEQ, _EMBED), jnp.float32),
            pltpu.SemaphoreType.DMA,
        ],
    )
    return f(x32, token_table, pos_table)
